# baseline (device time: 80940 ns/iter reference)
import functools

import jax
import jax.numpy as jnp
from jax import lax
from jax.experimental import pallas as pl
from jax.experimental.pallas import tpu as pltpu

N_DEV = 4
N_HOP = N_DEV - 1
S = 4


def kernel(x, w_mat):
    m, k_per = x.shape
    k_per2, n = w_mat.shape
    assert k_per == k_per2
    m_per = m // N_DEV
    n_half = n // 2
    n_sub = n_half // S

    def body(
        x_ref, w_ref, out_ref,
        seed_r_ref, seed_l_ref, recv_r_ref, recv_l_ref,
        send_r_sems, recv_r_sems, send_l_sems, recv_l_sems,
    ):
        me = lax.axis_index("i")
        left = lax.rem(me + N_DEV - 1, N_DEV)
        right = lax.rem(me + 1, N_DEV)

        def partial_r(c, s):
            return jnp.dot(
                x_ref[pl.ds(c * m_per, m_per), :],
                w_ref[:, s * n_sub:(s + 1) * n_sub],
                preferred_element_type=jnp.float32,
            )

        def partial_l(c, s):
            return jnp.dot(
                x_ref[pl.ds(c * m_per, m_per), :],
                w_ref[:, n_half + s * n_sub:n_half + (s + 1) * n_sub],
                preferred_element_type=jnp.float32,
            )

        def fwd(src_ref, dst_slot_ref, send_sem, recv_sem, dev):
            d = pltpu.make_async_remote_copy(
                src_ref=src_ref, dst_ref=dst_slot_ref,
                send_sem=send_sem, recv_sem=recv_sem,
                device_id=(dev,), device_id_type=pl.DeviceIdType.MESH,
            )
            d.start()
            return d

        c_seed_r = lax.rem(me + N_DEV - 1, N_DEV)
        c_seed_l = lax.rem(me + 1, N_DEV)
        for s in range(S):
            seed_r_ref[s] = partial_r(c_seed_r, s)
            seed_l_ref[s] = partial_l(c_seed_l, s)

        barrier_sem = pltpu.get_barrier_semaphore()
        for nbr in (left, right):
            pl.semaphore_signal(
                barrier_sem, inc=1,
                device_id=(nbr,), device_id_type=pl.DeviceIdType.MESH,
            )
        pl.semaphore_wait(barrier_sem, 2)

        sends = []
        for s in range(S):
            sends.append(fwd(
                seed_r_ref.at[s], recv_r_ref.at[0, s],
                send_r_sems.at[0, s], recv_r_sems.at[0, s], right,
            ))
            sends.append(fwd(
                seed_l_ref.at[s], recv_l_ref.at[0, s],
                send_l_sems.at[0, s], recv_l_sems.at[0, s], left,
            ))

        for h in range(N_HOP):
            c_r = lax.rem(me + 2 * N_DEV - 2 - h, N_DEV)
            c_l = lax.rem(me + 2 + h, N_DEV)
            for s in range(S):
                pr = partial_r(c_r, s)
                plf = partial_l(c_l, s)

                rv_r = pltpu.make_async_remote_copy(
                    src_ref=recv_r_ref.at[h, s], dst_ref=recv_r_ref.at[h, s],
                    send_sem=send_r_sems.at[h, s],
                    recv_sem=recv_r_sems.at[h, s],
                    device_id=(right,), device_id_type=pl.DeviceIdType.MESH,
                )
                rv_r.wait_recv()
                if h < N_HOP - 1:
                    recv_r_ref[h, s] = recv_r_ref[h, s] + pr
                    sends.append(fwd(
                        recv_r_ref.at[h, s], recv_r_ref.at[h + 1, s],
                        send_r_sems.at[h + 1, s], recv_r_sems.at[h + 1, s],
                        right,
                    ))
                else:
                    out_ref[:, s * n_sub:(s + 1) * n_sub] = jnp.maximum(
                        recv_r_ref[h, s] + pr, 0.0
                    )

                rv_l = pltpu.make_async_remote_copy(
                    src_ref=recv_l_ref.at[h, s], dst_ref=recv_l_ref.at[h, s],
                    send_sem=send_l_sems.at[h, s],
                    recv_sem=recv_l_sems.at[h, s],
                    device_id=(left,), device_id_type=pl.DeviceIdType.MESH,
                )
                rv_l.wait_recv()
                if h < N_HOP - 1:
                    recv_l_ref[h, s] = recv_l_ref[h, s] + plf
                    sends.append(fwd(
                        recv_l_ref.at[h, s], recv_l_ref.at[h + 1, s],
                        send_l_sems.at[h + 1, s], recv_l_sems.at[h + 1, s],
                        left,
                    ))
                else:
                    lo = n_half + s * n_sub
                    out_ref[:, lo:lo + n_sub] = jnp.maximum(
                        recv_l_ref[h, s] + plf, 0.0
                    )

        for d in sends:
            d.wait_send()

        @functools.partial(
            pl.run_scoped, second_barrier=pltpu.SemaphoreType.REGULAR
        )
        def _(second_barrier):
            for nbr in (left, right):
                pl.semaphore_signal(
                    second_barrier, inc=1,
                    device_id=(nbr,), device_id_type=pl.DeviceIdType.MESH,
                )
            pl.semaphore_wait(second_barrier, 2)

    return pl.pallas_call(
        body,
        out_shape=jax.ShapeDtypeStruct((m_per, n), jnp.float32),
        in_specs=[
            pl.BlockSpec(memory_space=pltpu.VMEM),
            pl.BlockSpec(memory_space=pltpu.VMEM),
        ],
        out_specs=pl.BlockSpec(memory_space=pltpu.VMEM),
        scratch_shapes=[
            pltpu.VMEM((S, m_per, n_sub), jnp.float32),
            pltpu.VMEM((S, m_per, n_sub), jnp.float32),
            pltpu.VMEM((N_HOP, S, m_per, n_sub), jnp.float32),
            pltpu.VMEM((N_HOP, S, m_per, n_sub), jnp.float32),
            pltpu.SemaphoreType.DMA((N_HOP, S)),
            pltpu.SemaphoreType.DMA((N_HOP, S)),
            pltpu.SemaphoreType.DMA((N_HOP, S)),
            pltpu.SemaphoreType.DMA((N_HOP, S)),
        ],
        compiler_params=pltpu.CompilerParams(collective_id=0),
    )(x, w_mat)


# device time: 80306 ns/iter; 1.0079x vs baseline; 1.0079x over previous
import functools

import jax
import jax.numpy as jnp
from jax import lax
from jax.experimental import pallas as pl
from jax.experimental.pallas import tpu as pltpu

N_DEV = 4
N_HOP = N_DEV - 1
S = 4


def kernel(x, w_mat):
    m, k_per = x.shape
    k_per2, n = w_mat.shape
    assert k_per == k_per2
    m_per = m // N_DEV
    n_half = n // 2
    n_sub = n_half // S

    def body(
        x_ref, w_ref, out_ref,
        seed_r_ref, seed_l_ref, recv_r_ref, recv_l_ref,
        send_r_sems, recv_r_sems, send_l_sems, recv_l_sems,
    ):
        me = lax.axis_index("i")
        left = lax.rem(me + N_DEV - 1, N_DEV)
        right = lax.rem(me + 1, N_DEV)

        def partial_r(c, s):
            return jnp.dot(
                x_ref[pl.ds(c * m_per, m_per), :],
                w_ref[:, s * n_sub:(s + 1) * n_sub],
                preferred_element_type=jnp.float32,
            )

        def partial_l(c, s):
            return jnp.dot(
                x_ref[pl.ds(c * m_per, m_per), :],
                w_ref[:, n_half + s * n_sub:n_half + (s + 1) * n_sub],
                preferred_element_type=jnp.float32,
            )

        def fwd(src_ref, dst_slot_ref, send_sem, recv_sem, dev):
            d = pltpu.make_async_remote_copy(
                src_ref=src_ref, dst_ref=dst_slot_ref,
                send_sem=send_sem, recv_sem=recv_sem,
                device_id=(dev,), device_id_type=pl.DeviceIdType.MESH,
            )
            d.start()
            return d

        barrier_sem = pltpu.get_barrier_semaphore()
        for nbr in (left, right):
            pl.semaphore_signal(
                barrier_sem, inc=1,
                device_id=(nbr,), device_id_type=pl.DeviceIdType.MESH,
            )
        pl.semaphore_wait(barrier_sem, 2)

        sends = []

        c_seed_r = lax.rem(me + N_DEV - 1, N_DEV)
        c_seed_l = lax.rem(me + 1, N_DEV)
        for s in range(S):
            seed_r_ref[s] = partial_r(c_seed_r, s)
            sends.append(fwd(
                seed_r_ref.at[s], recv_r_ref.at[0, s],
                send_r_sems.at[0, s], recv_r_sems.at[0, s], right,
            ))
            seed_l_ref[s] = partial_l(c_seed_l, s)
            sends.append(fwd(
                seed_l_ref.at[s], recv_l_ref.at[0, s],
                send_l_sems.at[0, s], recv_l_sems.at[0, s], left,
            ))

        for h in range(N_HOP):
            c_r = lax.rem(me + 2 * N_DEV - 2 - h, N_DEV)
            c_l = lax.rem(me + 2 + h, N_DEV)
            for s in range(S):
                pr = partial_r(c_r, s)
                plf = partial_l(c_l, s)

                rv_r = pltpu.make_async_remote_copy(
                    src_ref=recv_r_ref.at[h, s], dst_ref=recv_r_ref.at[h, s],
                    send_sem=send_r_sems.at[h, s],
                    recv_sem=recv_r_sems.at[h, s],
                    device_id=(right,), device_id_type=pl.DeviceIdType.MESH,
                )
                rv_r.wait_recv()
                if h < N_HOP - 1:
                    recv_r_ref[h, s] = recv_r_ref[h, s] + pr
                    sends.append(fwd(
                        recv_r_ref.at[h, s], recv_r_ref.at[h + 1, s],
                        send_r_sems.at[h + 1, s], recv_r_sems.at[h + 1, s],
                        right,
                    ))
                else:
                    out_ref[:, s * n_sub:(s + 1) * n_sub] = jnp.maximum(
                        recv_r_ref[h, s] + pr, 0.0
                    )

                rv_l = pltpu.make_async_remote_copy(
                    src_ref=recv_l_ref.at[h, s], dst_ref=recv_l_ref.at[h, s],
                    send_sem=send_l_sems.at[h, s],
                    recv_sem=recv_l_sems.at[h, s],
                    device_id=(left,), device_id_type=pl.DeviceIdType.MESH,
                )
                rv_l.wait_recv()
                if h < N_HOP - 1:
                    recv_l_ref[h, s] = recv_l_ref[h, s] + plf
                    sends.append(fwd(
                        recv_l_ref.at[h, s], recv_l_ref.at[h + 1, s],
                        send_l_sems.at[h + 1, s], recv_l_sems.at[h + 1, s],
                        left,
                    ))
                else:
                    lo = n_half + s * n_sub
                    out_ref[:, lo:lo + n_sub] = jnp.maximum(
                        recv_l_ref[h, s] + plf, 0.0
                    )

        for d in sends:
            d.wait_send()

        @functools.partial(
            pl.run_scoped, second_barrier=pltpu.SemaphoreType.REGULAR
        )
        def _(second_barrier):
            for nbr in (left, right):
                pl.semaphore_signal(
                    second_barrier, inc=1,
                    device_id=(nbr,), device_id_type=pl.DeviceIdType.MESH,
                )
            pl.semaphore_wait(second_barrier, 2)

    return pl.pallas_call(
        body,
        out_shape=jax.ShapeDtypeStruct((m_per, n), jnp.float32),
        in_specs=[
            pl.BlockSpec(memory_space=pltpu.VMEM),
            pl.BlockSpec(memory_space=pltpu.VMEM),
        ],
        out_specs=pl.BlockSpec(memory_space=pltpu.VMEM),
        scratch_shapes=[
            pltpu.VMEM((S, m_per, n_sub), jnp.float32),
            pltpu.VMEM((S, m_per, n_sub), jnp.float32),
            pltpu.VMEM((N_HOP, S, m_per, n_sub), jnp.float32),
            pltpu.VMEM((N_HOP, S, m_per, n_sub), jnp.float32),
            pltpu.SemaphoreType.DMA((N_HOP, S)),
            pltpu.SemaphoreType.DMA((N_HOP, S)),
            pltpu.SemaphoreType.DMA((N_HOP, S)),
            pltpu.SemaphoreType.DMA((N_HOP, S)),
        ],
        compiler_params=pltpu.CompilerParams(collective_id=0),
    )(x, w_mat)


# device time: 46478 ns/iter; 1.7415x vs baseline; 1.7278x over previous
import functools

import jax
import jax.numpy as jnp
from jax import lax
from jax.experimental import pallas as pl
from jax.experimental.pallas import tpu as pltpu

N_DEV = 4
N_HOP = N_DEV - 1
S = 4


def kernel(x, w_mat):
    m, k_per = x.shape
    k_per2, n = w_mat.shape
    assert k_per == k_per2
    m_per = m // N_DEV
    n_half = n // 2
    n_sub = n_half // S

    def body(
        x_ref, w_ref, out_ref,
        seed_r_ref, seed_l_ref, recv_r_ref, recv_l_ref,
        send_r_sems, recv_r_sems, send_l_sems, recv_l_sems,
    ):
        me = lax.axis_index("i")
        left = lax.rem(me + N_DEV - 1, N_DEV)
        right = lax.rem(me + 1, N_DEV)

        def partial_r(c, s):
            return jnp.dot(
                x_ref[pl.ds(c * m_per, m_per), :],
                w_ref[:, s * n_sub:(s + 1) * n_sub],
                preferred_element_type=jnp.float32,
            )

        def partial_l(c, s):
            return jnp.dot(
                x_ref[pl.ds(c * m_per, m_per), :],
                w_ref[:, n_half + s * n_sub:n_half + (s + 1) * n_sub],
                preferred_element_type=jnp.float32,
            )

        def fwd(src_ref, dst_slot_ref, send_sem, recv_sem, dev):
            d = pltpu.make_async_remote_copy(
                src_ref=src_ref, dst_ref=dst_slot_ref,
                send_sem=send_sem, recv_sem=recv_sem,
                device_id=(dev,), device_id_type=pl.DeviceIdType.MESH,
            )
            d.start()
            return d

        barrier_sem = pltpu.get_barrier_semaphore()
        for nbr in (left, right):
            pl.semaphore_signal(
                barrier_sem, inc=1,
                device_id=(nbr,), device_id_type=pl.DeviceIdType.MESH,
            )
        pl.semaphore_wait(barrier_sem, 2)

        sends = []

        c_seed_r = lax.rem(me + N_DEV - 1, N_DEV)
        c_seed_l = lax.rem(me + 1, N_DEV)
        for s in range(S):
            seed_r_ref[s] = partial_r(c_seed_r, s).astype(jnp.bfloat16)
            sends.append(fwd(
                seed_r_ref.at[s], recv_r_ref.at[0, s],
                send_r_sems.at[0, s], recv_r_sems.at[0, s], right,
            ))
            seed_l_ref[s] = partial_l(c_seed_l, s).astype(jnp.bfloat16)
            sends.append(fwd(
                seed_l_ref.at[s], recv_l_ref.at[0, s],
                send_l_sems.at[0, s], recv_l_sems.at[0, s], left,
            ))

        for h in range(N_HOP):
            c_r = lax.rem(me + 2 * N_DEV - 2 - h, N_DEV)
            c_l = lax.rem(me + 2 + h, N_DEV)
            for s in range(S):
                pr = partial_r(c_r, s)
                plf = partial_l(c_l, s)

                rv_r = pltpu.make_async_remote_copy(
                    src_ref=recv_r_ref.at[h, s], dst_ref=recv_r_ref.at[h, s],
                    send_sem=send_r_sems.at[h, s],
                    recv_sem=recv_r_sems.at[h, s],
                    device_id=(right,), device_id_type=pl.DeviceIdType.MESH,
                )
                rv_r.wait_recv()
                if h < N_HOP - 1:
                    recv_r_ref[h, s] = (
                        recv_r_ref[h, s].astype(jnp.float32) + pr
                    ).astype(jnp.bfloat16)
                    sends.append(fwd(
                        recv_r_ref.at[h, s], recv_r_ref.at[h + 1, s],
                        send_r_sems.at[h + 1, s], recv_r_sems.at[h + 1, s],
                        right,
                    ))
                else:
                    out_ref[:, s * n_sub:(s + 1) * n_sub] = jnp.maximum(
                        recv_r_ref[h, s].astype(jnp.float32) + pr, 0.0
                    )

                rv_l = pltpu.make_async_remote_copy(
                    src_ref=recv_l_ref.at[h, s], dst_ref=recv_l_ref.at[h, s],
                    send_sem=send_l_sems.at[h, s],
                    recv_sem=recv_l_sems.at[h, s],
                    device_id=(left,), device_id_type=pl.DeviceIdType.MESH,
                )
                rv_l.wait_recv()
                if h < N_HOP - 1:
                    recv_l_ref[h, s] = (
                        recv_l_ref[h, s].astype(jnp.float32) + plf
                    ).astype(jnp.bfloat16)
                    sends.append(fwd(
                        recv_l_ref.at[h, s], recv_l_ref.at[h + 1, s],
                        send_l_sems.at[h + 1, s], recv_l_sems.at[h + 1, s],
                        left,
                    ))
                else:
                    lo = n_half + s * n_sub
                    out_ref[:, lo:lo + n_sub] = jnp.maximum(
                        recv_l_ref[h, s].astype(jnp.float32) + plf, 0.0
                    )

        for d in sends:
            d.wait_send()

        @functools.partial(
            pl.run_scoped, second_barrier=pltpu.SemaphoreType.REGULAR
        )
        def _(second_barrier):
            for nbr in (left, right):
                pl.semaphore_signal(
                    second_barrier, inc=1,
                    device_id=(nbr,), device_id_type=pl.DeviceIdType.MESH,
                )
            pl.semaphore_wait(second_barrier, 2)

    return pl.pallas_call(
        body,
        out_shape=jax.ShapeDtypeStruct((m_per, n), jnp.float32),
        in_specs=[
            pl.BlockSpec(memory_space=pltpu.VMEM),
            pl.BlockSpec(memory_space=pltpu.VMEM),
        ],
        out_specs=pl.BlockSpec(memory_space=pltpu.VMEM),
        scratch_shapes=[
            pltpu.VMEM((S, m_per, n_sub), jnp.bfloat16),
            pltpu.VMEM((S, m_per, n_sub), jnp.bfloat16),
            pltpu.VMEM((N_HOP, S, m_per, n_sub), jnp.bfloat16),
            pltpu.VMEM((N_HOP, S, m_per, n_sub), jnp.bfloat16),
            pltpu.SemaphoreType.DMA((N_HOP, S)),
            pltpu.SemaphoreType.DMA((N_HOP, S)),
            pltpu.SemaphoreType.DMA((N_HOP, S)),
            pltpu.SemaphoreType.DMA((N_HOP, S)),
        ],
        compiler_params=pltpu.CompilerParams(collective_id=0),
    )(x, w_mat)
